# Initial kernel scaffold; baseline (speedup 1.0000x reference)
import jax
import jax.numpy as jnp
from jax.experimental import pallas as pl


def kernel(queries, keys, values):
    raise NotImplementedError("write your pallas kernel here")


# speedup vs baseline: 3.3792x; 3.3792x over previous
import functools

import jax
import jax.numpy as jnp
from jax.experimental import pallas as pl
from jax.experimental.pallas import tpu as pltpu

K_NN = 32


def _knn_kernel(q_ref, k_ref, v_ref, o_ref, topd_ref, topv_ref,
                lm_ref, li_ref, done_ref, *, n_keys, chunk, n_chunks):
    c = pl.program_id(0)

    @pl.when(c == 0)
    def _init():
        topd_ref[...] = jnp.full_like(topd_ref, jnp.inf)
        topv_ref[...] = jnp.zeros_like(topv_ref)

    q = q_ref[...]
    k = k_ref[...]
    vb = v_ref[0]

    qsq = jnp.sum(q * q, axis=1, keepdims=True)
    ksq = jnp.sum(k * k, axis=1)[None, :]
    qk = jax.lax.dot_general(q, k, (((1,), (1,)), ((), ())),
                             preferred_element_type=jnp.float32)
    dist = qsq - 2.0 * qk + ksq

    lane = jax.lax.broadcasted_iota(jnp.int32, (1, chunk), 1).astype(jnp.float32)
    valid = (jnp.float32(c * chunk) + lane) < n_keys
    dist = jnp.where(valid, dist, jnp.inf)

    slot = jax.lax.broadcasted_iota(jnp.int32, topd_ref.shape, 1).astype(jnp.float32)

    lm_ref[...] = jnp.full_like(lm_ref, -jnp.inf)
    li_ref[...] = jnp.full_like(li_ref, -1.0)
    done_ref[0] = 0

    def body(j, _):
        @pl.when(done_ref[0] == 0)
        def _step():
            lm = lm_ref[...]
            li = li_ref[...]
            lex = (dist > lm) | ((dist == lm) & (lane > li))
            m = jnp.min(jnp.where(lex, dist, jnp.inf), axis=1, keepdims=True)
            topd = topd_ref[...]
            t = jnp.max(topd, axis=1, keepdims=True)
            accept = m < t
            n_acc = jnp.sum(accept.astype(jnp.float32))

            @pl.when(n_acc == 0.0)
            def _done():
                done_ref[0] = 1

            @pl.when(n_acc > 0.0)
            def _merge():
                idx = jnp.min(jnp.where((dist == m) & lex, lane, jnp.float32(2 ** 30)),
                              axis=1, keepdims=True)
                val = jnp.sum(jnp.where(lane == idx, vb, 0.0), axis=1, keepdims=True)
                pos = jnp.min(jnp.where(topd == t, slot, jnp.float32(64.0)),
                              axis=1, keepdims=True)
                repl = (slot == pos) & accept
                topd_ref[...] = jnp.where(repl, m, topd)
                topv_ref[...] = jnp.where(repl, val, topv_ref[...])
                lm_ref[...] = m
                li_ref[...] = idx

        return 0

    jax.lax.fori_loop(0, K_NN, body, 0)

    @pl.when(c == n_chunks - 1)
    def _emit():
        o_ref[...] = jnp.sum(topv_ref[...], axis=1, keepdims=True) / jnp.float32(K_NN)


def kernel(queries, keys, values):
    n_queries, dim = queries.shape
    n_keys = keys.shape[0]
    chunk = 2048
    n_chunks = pl.cdiv(n_keys, chunk)
    n_pad = n_chunks * chunk

    keys_p = jnp.pad(keys, ((0, n_pad - n_keys), (0, 0)))
    values_p = jnp.pad(values, (0, n_pad - n_keys)).reshape(n_chunks, 1, chunk)

    out = pl.pallas_call(
        functools.partial(_knn_kernel, n_keys=n_keys, chunk=chunk,
                          n_chunks=n_chunks),
        grid=(n_chunks,),
        in_specs=[
            pl.BlockSpec((n_queries, dim), lambda c: (0, 0)),
            pl.BlockSpec((chunk, dim), lambda c: (c, 0)),
            pl.BlockSpec((1, 1, chunk), lambda c: (c, 0, 0)),
        ],
        out_specs=pl.BlockSpec((n_queries, 1), lambda c: (0, 0)),
        out_shape=jax.ShapeDtypeStruct((n_queries, 1), jnp.float32),
        scratch_shapes=[
            pltpu.VMEM((n_queries, K_NN), jnp.float32),
            pltpu.VMEM((n_queries, K_NN), jnp.float32),
            pltpu.VMEM((n_queries, 1), jnp.float32),
            pltpu.VMEM((n_queries, 1), jnp.float32),
            pltpu.SMEM((1,), jnp.int32),
        ],
    )(queries, keys_p, values_p)
    return out[:, 0]
